# Initial kernel scaffold; baseline (speedup 1.0000x reference)
import jax
import jax.numpy as jnp
from jax.experimental import pallas as pl


def kernel(asc_dec, cru_dec, des_dec, concat_index):
    raise NotImplementedError("write your pallas kernel here")


# speedup vs baseline: 1.0115x; 1.0115x over previous
import functools

import jax
import jax.numpy as jnp
from jax import lax
from jax.experimental import pallas as pl
from jax.experimental.pallas import tpu as pltpu
from jax.experimental.pallas import tpu_sc as plsc

N = 12288
NSRC = 3
SRC_ROWS = 4096
D = 1024
Q = 96
P = 128
NB = 128

W = 32
RPW = SRC_ROWS // W
CH = 32
NCH = RPW // CH


def _rank_kernel(vcol_ref, vrow_ref, out_ref, f2_ref, macc_ref, gacc_ref):
    iota_sub = lax.broadcasted_iota(jnp.int32, (P, NB), 0)
    iota_lane = lax.broadcasted_iota(jnp.int32, (P, NB), 1)

    macc_ref[...] = jnp.zeros((NB, NB), jnp.float32)

    def loop1(q, carry):
        vcol = vcol_ref[pl.ds(q * P, P), :]
        vrow = vrow_ref[pl.ds(q, 1), :]
        l_col = jnp.bitwise_and(vcol, 127)
        h_row = jnp.right_shift(vrow, 7)
        oht = (h_row == iota_sub).astype(jnp.bfloat16)
        ol = (iota_lane == l_col).astype(jnp.bfloat16)
        f2 = lax.dot_general(oht, ol, (((1,), (0,)), ((), ())),
                             preferred_element_type=jnp.float32)
        macc_ref[...] += f2
        f2_ref[pl.ds(q * P, P), :] = f2.astype(jnp.bfloat16)
        return carry

    lax.fori_loop(0, Q, loop1, 0)

    macc = macc_ref[...]
    cnt_h = jnp.sum(macc, axis=1, keepdims=True)
    lt_bb = (iota_sub < iota_lane).astype(jnp.float32)
    sh_row = jnp.sum(cnt_h * lt_bb, axis=0, keepdims=True)
    m_hi = jnp.floor(macc * (1.0 / 256.0))
    m_lo = macc - m_hi * 256.0
    m_hi_bf = m_hi.astype(jnp.bfloat16)
    m_lo_bf = m_lo.astype(jnp.bfloat16)

    gacc_ref[...] = jnp.zeros((NB, NB), jnp.float32)

    def loop2(q, carry):
        vcol = vcol_ref[pl.ds(q * P, P), :]
        vrow = vrow_ref[pl.ds(q, 1), :]
        h_col = jnp.right_shift(vcol, 7)
        l_col = jnp.bitwise_and(vcol, 127)
        oh_bool = (h_col == iota_lane)
        oh_bf = oh_bool.astype(jnp.bfloat16)
        oh_f = oh_bool.astype(jnp.float32)
        a_col = jnp.sum(oh_f * sh_row, axis=1, keepdims=True)
        x = (lax.dot_general(oh_bf, m_hi_bf, (((1,), (0,)), ((), ())),
                             preferred_element_type=jnp.float32) * 256.0
             + lax.dot_general(oh_bf, m_lo_bf, (((1,), (0,)), ((), ())),
                               preferred_element_type=jnp.float32))
        b_col = jnp.sum(jnp.where(iota_lane < l_col, x, 0.0), axis=1,
                        keepdims=True)
        g = gacc_ref[...]
        g_hi = jnp.floor(g * (1.0 / 256.0))
        g_lo = g - g_hi * 256.0
        y = (lax.dot_general(oh_bf, g_hi.astype(jnp.bfloat16),
                             (((1,), (0,)), ((), ())),
                             preferred_element_type=jnp.float32) * 256.0
             + lax.dot_general(oh_bf, g_lo.astype(jnp.bfloat16),
                               (((1,), (0,)), ((), ())),
                               preferred_element_type=jnp.float32))
        ol_mask = (iota_lane == l_col)
        c1_col = jnp.sum(jnp.where(ol_mask, y, 0.0), axis=1, keepdims=True)
        eq = (vcol == vrow)
        earlier = iota_lane < iota_sub
        c2_col = jnp.sum(jnp.where(eq & earlier, 1.0, 0.0), axis=1,
                         keepdims=True)
        rank = a_col + b_col + c1_col + c2_col
        out_ref[pl.ds(q * P, P), :] = rank.astype(jnp.int32)
        gacc_ref[...] += f2_ref[pl.ds(q * P, P), :].astype(jnp.float32)
        return carry

    lax.fori_loop(0, Q, loop2, 0)


def _compute_rank(vcol, vrow):
    return pl.pallas_call(
        _rank_kernel,
        out_shape=jax.ShapeDtypeStruct((N, 1), jnp.int32),
        scratch_shapes=[
            pltpu.VMEM((N, NB), jnp.bfloat16),
            pltpu.VMEM((NB, NB), jnp.float32),
            pltpu.VMEM((NB, NB), jnp.float32),
        ],
    )(vcol, vrow)


def _sc_scatter(asc, cru, des, rank2d):
    mesh = plsc.VectorSubcoreMesh(core_axis_name="c", subcore_axis_name="s")

    @functools.partial(
        pl.kernel,
        out_type=jax.ShapeDtypeStruct((N, D), jnp.float32),
        mesh=mesh,
        scratch_types=[
            pltpu.VMEM((NCH, CH), jnp.int32),
            pltpu.VMEM((CH, D), jnp.float32),
            pltpu.VMEM((CH, D), jnp.float32),
        ],
    )
    def scatter_kernel(asc_hbm, cru_hbm, des_hbm, rank_hbm, out_hbm,
                       idx_v, buf_a, buf_b):
        wid = lax.axis_index("s") * 2 + lax.axis_index("c")
        row0 = wid * RPW
        for s, src in enumerate((asc_hbm, cru_hbm, des_hbm)):
            pltpu.sync_copy(
                rank_hbm.at[pl.ds(s * (SRC_ROWS // CH) + wid * NCH, NCH)],
                idx_v)
            for k in range(NCH):
                buf = buf_a if k % 2 == 0 else buf_b
                pltpu.sync_copy(src.at[pl.ds(row0 + k * CH, CH)], buf)
                pltpu.sync_copy(buf, out_hbm.at[idx_v.at[k]])

    return scatter_kernel(asc, cru, des, rank2d)


def kernel(asc_dec, cru_dec, des_dec, concat_index):
    v = concat_index.astype(jnp.int32)
    vcol = v.reshape(N, 1)
    vrow = v.reshape(Q, P)
    rank = _compute_rank(vcol, vrow)
    rank2d = rank.reshape(N // CH, CH)
    return _sc_scatter(asc_dec, cru_dec, des_dec, rank2d)


# speedup vs baseline: 1.3634x; 1.3479x over previous
import functools

import jax
import jax.numpy as jnp
from jax import lax
from jax.experimental import pallas as pl
from jax.experimental.pallas import tpu as pltpu
from jax.experimental.pallas import tpu_sc as plsc

N = 12288
NSRC = 3
SRC_ROWS = 4096
D = 1024
P = 512
Q = N // P
NB = 128

W = 32
RPW = SRC_ROWS // W
CH = 32
NCH = RPW // CH
NIT = NSRC * NCH


def _rank_kernel(vcol_ref, vrow_ref, out_ref, f2_ref, macc_ref, gacc_ref):
    iota_b_bp = lax.broadcasted_iota(jnp.int32, (NB, P), 0)
    iota_m_pm = lax.broadcasted_iota(jnp.int32, (P, NB), 1)
    iota_p_pp = lax.broadcasted_iota(jnp.int32, (P, P), 0)
    iota_j_pp = lax.broadcasted_iota(jnp.int32, (P, P), 1)

    macc_ref[...] = jnp.zeros((NB, NB), jnp.float32)

    def loop1(q, carry):
        vcol = vcol_ref[pl.ds(q * P, P), :]
        vrow = vrow_ref[pl.ds(q, 1), :]
        l_col = jnp.bitwise_and(vcol, 127)
        h_row = jnp.right_shift(vrow, 7)
        oht = (h_row == iota_b_bp).astype(jnp.bfloat16)
        ol = (iota_m_pm == l_col).astype(jnp.bfloat16)
        f2 = lax.dot_general(oht, ol, (((1,), (0,)), ((), ())),
                             preferred_element_type=jnp.float32)
        macc_ref[...] += f2
        f2_ref[pl.ds(q * NB, NB), :] = f2.astype(jnp.bfloat16)
        return carry

    lax.fori_loop(0, Q, loop1, 0)

    macc = macc_ref[...]
    cnt_h = jnp.sum(macc, axis=1, keepdims=True)
    iota_bb0 = lax.broadcasted_iota(jnp.int32, (NB, NB), 0)
    iota_bb1 = lax.broadcasted_iota(jnp.int32, (NB, NB), 1)
    lt_bb = (iota_bb0 < iota_bb1).astype(jnp.float32)
    sh_row = jnp.sum(cnt_h * lt_bb, axis=0, keepdims=True)
    m_hi = jnp.floor(macc * (1.0 / 256.0))
    m_lo = macc - m_hi * 256.0
    m_hi_bf = m_hi.astype(jnp.bfloat16)
    m_lo_bf = m_lo.astype(jnp.bfloat16)

    gacc_ref[...] = jnp.zeros((NB, NB), jnp.float32)

    def loop2(q, carry):
        vcol = vcol_ref[pl.ds(q * P, P), :]
        vrow = vrow_ref[pl.ds(q, 1), :]
        h_col = jnp.right_shift(vcol, 7)
        l_col = jnp.bitwise_and(vcol, 127)
        oh_bool = (h_col == iota_m_pm)
        oh_bf = oh_bool.astype(jnp.bfloat16)
        oh_f = oh_bool.astype(jnp.float32)
        a_col = jnp.sum(oh_f * sh_row, axis=1, keepdims=True)
        g = gacc_ref[...]
        g_hi = jnp.floor(g * (1.0 / 256.0))
        g_lo = g - g_hi * 256.0
        tbl = jnp.concatenate(
            [m_hi_bf, m_lo_bf,
             g_hi.astype(jnp.bfloat16), g_lo.astype(jnp.bfloat16)], axis=1)
        xy = lax.dot_general(oh_bf, tbl, (((1,), (0,)), ((), ())),
                             preferred_element_type=jnp.float32)
        x = xy[:, 0:NB] * 256.0 + xy[:, NB:2 * NB]
        y = xy[:, 2 * NB:3 * NB] * 256.0 + xy[:, 3 * NB:4 * NB]
        b_col = jnp.sum(jnp.where(iota_m_pm < l_col, x, 0.0), axis=1,
                        keepdims=True)
        c1_col = jnp.sum(jnp.where(iota_m_pm == l_col, y, 0.0), axis=1,
                         keepdims=True)
        eq = (vcol == vrow)
        c2_col = jnp.sum(jnp.where(eq & (iota_j_pp < iota_p_pp), 1.0, 0.0),
                         axis=1, keepdims=True)
        rank = a_col + b_col + c1_col + c2_col
        out_ref[pl.ds(q * P, P), :] = rank.astype(jnp.int32)
        gacc_ref[...] += f2_ref[pl.ds(q * NB, NB), :].astype(jnp.float32)
        return carry

    lax.fori_loop(0, Q, loop2, 0)


def _compute_rank(vcol, vrow):
    return pl.pallas_call(
        _rank_kernel,
        out_shape=jax.ShapeDtypeStruct((N, 1), jnp.int32),
        scratch_shapes=[
            pltpu.VMEM((Q * NB, NB), jnp.bfloat16),
            pltpu.VMEM((NB, NB), jnp.float32),
            pltpu.VMEM((NB, NB), jnp.float32),
        ],
    )(vcol, vrow)


def _sc_scatter(asc, cru, des, rank2d):
    mesh = plsc.VectorSubcoreMesh(core_axis_name="c", subcore_axis_name="s")

    @functools.partial(
        pl.kernel,
        out_type=jax.ShapeDtypeStruct((N, D), jnp.float32),
        mesh=mesh,
        scratch_types=[
            pltpu.VMEM((NIT, CH), jnp.int32),
            pltpu.VMEM((CH, D), jnp.float32),
            pltpu.VMEM((CH, D), jnp.float32),
            pltpu.SemaphoreType.DMA,
            pltpu.SemaphoreType.DMA,
            pltpu.SemaphoreType.DMA,
            pltpu.SemaphoreType.DMA,
        ],
    )
    def scatter_kernel(asc_hbm, cru_hbm, des_hbm, rank_hbm, out_hbm,
                       idx_v, buf_a, buf_b, lsem_a, lsem_b, ssem_a, ssem_b):
        wid = lax.axis_index("s") * 2 + lax.axis_index("c")
        row0 = wid * RPW
        srcs = (asc_hbm, cru_hbm, des_hbm)
        bufs = (buf_a, buf_b)
        lsems = (lsem_a, lsem_b)
        ssems = (ssem_a, ssem_b)
        for s in range(NSRC):
            pltpu.sync_copy(
                rank_hbm.at[pl.ds(s * (SRC_ROWS // CH) + wid * NCH, NCH)],
                idx_v.at[pl.ds(s * NCH, NCH)])

        def start_load(i):
            s, k = divmod(i, NCH)
            return pltpu.async_copy(
                srcs[s].at[pl.ds(row0 + k * CH, CH)], bufs[i % 2],
                lsems[i % 2])

        loads = {0: start_load(0)}
        stores = {}
        for i in range(NIT):
            if i + 1 < NIT:
                if i - 1 >= 0:
                    stores[i - 1].wait()
                loads[i + 1] = start_load(i + 1)
            loads[i].wait()
            stores[i] = pltpu.async_copy(
                bufs[i % 2], out_hbm.at[idx_v.at[i]], ssems[i % 2])
        stores[NIT - 2].wait()
        stores[NIT - 1].wait()

    return scatter_kernel(asc, cru, des, rank2d)


def kernel(asc_dec, cru_dec, des_dec, concat_index):
    v = concat_index.astype(jnp.int32)
    vcol = v.reshape(N, 1)
    vrow = v.reshape(Q, P)
    rank = _compute_rank(vcol, vrow)
    rank2d = rank.reshape(N // CH, CH)
    return _sc_scatter(asc_dec, cru_dec, des_dec, rank2d)


# speedup vs baseline: 1.7148x; 1.2577x over previous
import functools

import jax
import jax.numpy as jnp
from jax import lax
from jax.experimental import pallas as pl
from jax.experimental.pallas import tpu as pltpu
from jax.experimental.pallas import tpu_sc as plsc

N = 12288
NSRC = 3
SRC_ROWS = 4096
D = 1024
P = 512
Q = N // P
NB = 128

W = 32
RPW = SRC_ROWS // W
CH = 32
NCH = RPW // CH
NIT = NSRC * NCH


def _rank_kernel(vrow_ref, out_ref, f2t_ref, macc_ref, gacc_ref, c2_ref):
    iota_b_bP = lax.broadcasted_iota(jnp.int32, (NB, P), 0)
    iota_l_Pb = lax.broadcasted_iota(jnp.int32, (P, NB), 1)
    iota_p_PP = lax.broadcasted_iota(jnp.int32, (P, P), 0)
    iota_j_PP = lax.broadcasted_iota(jnp.int32, (P, P), 1)
    diag_PP = (iota_p_PP == iota_j_PP).astype(jnp.float32)

    macc_ref[...] = jnp.zeros((NB, NB), jnp.float32)

    def loop1(q, carry):
        vrow = vrow_ref[pl.ds(q, 1), :]
        vf = vrow.astype(jnp.float32)
        vcol = jnp.sum(vf * diag_PP, axis=1, keepdims=True).astype(jnp.int32)
        l_row = jnp.bitwise_and(vrow, 127)
        h_col = jnp.right_shift(vcol, 7)
        olt = (l_row == iota_b_bP).astype(jnp.bfloat16)
        oh = (h_col == iota_l_Pb).astype(jnp.bfloat16)
        f2t = lax.dot_general(olt, oh, (((1,), (0,)), ((), ())),
                              preferred_element_type=jnp.float32)
        macc_ref[...] += f2t
        f2t_ref[pl.ds(q * NB, NB), :] = f2t.astype(jnp.bfloat16)
        eq = (vcol == vrow) & (iota_j_PP < iota_p_PP)
        c2_col = jnp.sum(jnp.where(eq, 1.0, 0.0), axis=1, keepdims=True)
        c2_row = jnp.sum(c2_col * diag_PP, axis=0, keepdims=True)
        c2_ref[pl.ds(q, 1), :] = c2_row
        return carry

    lax.fori_loop(0, Q, loop1, 0)

    macct = macc_ref[...]
    cnt_h = jnp.sum(macct, axis=0, keepdims=True)
    iota_bb0 = lax.broadcasted_iota(jnp.int32, (NB, NB), 0)
    iota_bb1 = lax.broadcasted_iota(jnp.int32, (NB, NB), 1)
    lt_bb = (iota_bb1 < iota_bb0).astype(jnp.float32)
    sh_col = jnp.sum(cnt_h * lt_bb, axis=1, keepdims=True)
    m_hi = jnp.floor(macct * (1.0 / 256.0))
    m_lo = macct - m_hi * 256.0
    m_hi_bf = m_hi.astype(jnp.bfloat16)
    m_lo_bf = m_lo.astype(jnp.bfloat16)

    gacc_ref[...] = jnp.zeros((NB, NB), jnp.float32)

    def loop2(q, carry):
        vrow = vrow_ref[pl.ds(q, 1), :]
        h_row = jnp.right_shift(vrow, 7)
        l_row = jnp.bitwise_and(vrow, 127)
        oht_bool = (h_row == iota_b_bP)
        oht_bf = oht_bool.astype(jnp.bfloat16)
        oht_f = oht_bool.astype(jnp.float32)
        a_row = jnp.sum(oht_f * sh_col, axis=0, keepdims=True)
        g = gacc_ref[...]
        g_hi = jnp.floor(g * (1.0 / 256.0))
        g_lo = g - g_hi * 256.0
        tblt = jnp.concatenate(
            [m_hi_bf, m_lo_bf,
             g_hi.astype(jnp.bfloat16), g_lo.astype(jnp.bfloat16)], axis=0)
        xyt = lax.dot_general(tblt, oht_bf, (((1,), (0,)), ((), ())),
                              preferred_element_type=jnp.float32)
        xt = xyt[0:NB, :] * 256.0 + xyt[NB:2 * NB, :]
        yt = xyt[2 * NB:3 * NB, :] * 256.0 + xyt[3 * NB:4 * NB, :]
        b_row = jnp.sum(jnp.where(iota_b_bP < l_row, xt, 0.0), axis=0,
                        keepdims=True)
        c1_row = jnp.sum(jnp.where(iota_b_bP == l_row, yt, 0.0), axis=0,
                         keepdims=True)
        rank = a_row + b_row + c1_row + c2_ref[pl.ds(q, 1), :]
        out_ref[pl.ds(q, 1), :] = rank.astype(jnp.int32)
        gacc_ref[...] += f2t_ref[pl.ds(q * NB, NB), :].astype(jnp.float32)
        return carry

    lax.fori_loop(0, Q, loop2, 0)


def _compute_rank(vrow):
    return pl.pallas_call(
        _rank_kernel,
        out_shape=jax.ShapeDtypeStruct((Q, P), jnp.int32),
        scratch_shapes=[
            pltpu.VMEM((Q * NB, NB), jnp.bfloat16),
            pltpu.VMEM((NB, NB), jnp.float32),
            pltpu.VMEM((NB, NB), jnp.float32),
            pltpu.VMEM((Q, P), jnp.float32),
        ],
    )(vrow)


def _sc_scatter(asc, cru, des, rank2d):
    mesh = plsc.VectorSubcoreMesh(core_axis_name="c", subcore_axis_name="s")

    @functools.partial(
        pl.kernel,
        out_type=jax.ShapeDtypeStruct((N, D), jnp.float32),
        mesh=mesh,
        scratch_types=[
            pltpu.VMEM((NIT, CH), jnp.int32),
            pltpu.VMEM((CH, D), jnp.float32),
            pltpu.VMEM((CH, D), jnp.float32),
            pltpu.SemaphoreType.DMA,
            pltpu.SemaphoreType.DMA,
            pltpu.SemaphoreType.DMA,
            pltpu.SemaphoreType.DMA,
        ],
    )
    def scatter_kernel(asc_hbm, cru_hbm, des_hbm, rank_hbm, out_hbm,
                       idx_v, buf_a, buf_b, lsem_a, lsem_b, ssem_a, ssem_b):
        wid = lax.axis_index("s") * 2 + lax.axis_index("c")
        row0 = wid * RPW
        srcs = (asc_hbm, cru_hbm, des_hbm)
        bufs = (buf_a, buf_b)
        lsems = (lsem_a, lsem_b)
        ssems = (ssem_a, ssem_b)
        for s in range(NSRC):
            pltpu.sync_copy(
                rank_hbm.at[pl.ds(s * (SRC_ROWS // CH) + wid * NCH, NCH)],
                idx_v.at[pl.ds(s * NCH, NCH)])

        def start_load(i):
            s, k = divmod(i, NCH)
            return pltpu.async_copy(
                srcs[s].at[pl.ds(row0 + k * CH, CH)], bufs[i % 2],
                lsems[i % 2])

        loads = {0: start_load(0)}
        stores = {}
        for i in range(NIT):
            if i + 1 < NIT:
                if i - 1 >= 0:
                    stores[i - 1].wait()
                loads[i + 1] = start_load(i + 1)
            loads[i].wait()
            stores[i] = pltpu.async_copy(
                bufs[i % 2], out_hbm.at[idx_v.at[i]], ssems[i % 2])
        stores[NIT - 2].wait()
        stores[NIT - 1].wait()

    return scatter_kernel(asc, cru, des, rank2d)


def kernel(asc_dec, cru_dec, des_dec, concat_index):
    v = concat_index.astype(jnp.int32)
    vrow = v.reshape(Q, P)
    rank = _compute_rank(vrow)
    rank2d = rank.reshape(N // CH, CH)
    return _sc_scatter(asc_dec, cru_dec, des_dec, rank2d)


# speedup vs baseline: 1.7464x; 1.0184x over previous
import functools

import jax
import jax.numpy as jnp
from jax import lax
from jax.experimental import pallas as pl
from jax.experimental.pallas import tpu as pltpu
from jax.experimental.pallas import tpu_sc as plsc

N = 12288
NSRC = 3
SRC_ROWS = 4096
D = 1024
P = 512
Q = N // P
NB = 128

W = 32
RPW = SRC_ROWS // W
CH = 32
NCH = RPW // CH
NIT = NSRC * NCH


def _rank_kernel(vrow_ref, out_ref, f2t_ref, macc_ref, gacc_ref, c2_ref):
    iota_b_bP = lax.broadcasted_iota(jnp.int32, (NB, P), 0)
    iota_l_Pb = lax.broadcasted_iota(jnp.int32, (P, NB), 1)
    iota_p_PP = lax.broadcasted_iota(jnp.int32, (P, P), 0)
    iota_j_PP = lax.broadcasted_iota(jnp.int32, (P, P), 1)
    diag_PP = (iota_p_PP == iota_j_PP).astype(jnp.float32)

    macc_ref[...] = jnp.zeros((NB, NB), jnp.float32)

    def loop1(q, carry):
        vrow = vrow_ref[pl.ds(q, 1), :]
        vf = vrow.astype(jnp.float32)
        vcol = jnp.sum(vf * diag_PP, axis=1, keepdims=True).astype(jnp.int32)
        l_row = jnp.bitwise_and(vrow, 127)
        h_col = jnp.right_shift(vcol, 7)
        olt = (l_row == iota_b_bP).astype(jnp.bfloat16)
        oh = (h_col == iota_l_Pb).astype(jnp.bfloat16)
        f2t = lax.dot_general(olt, oh, (((1,), (0,)), ((), ())),
                              preferred_element_type=jnp.float32)
        macc_ref[...] += f2t
        f2t_ref[pl.ds(q * NB, NB), :] = f2t.astype(jnp.bfloat16)
        eq = (vcol == vrow) & (iota_j_PP < iota_p_PP)
        c2_col = jnp.sum(jnp.where(eq, 1.0, 0.0), axis=1, keepdims=True)
        c2_row = jnp.sum(c2_col * diag_PP, axis=0, keepdims=True)
        c2_ref[pl.ds(q, 1), :] = c2_row
        return carry

    lax.fori_loop(0, Q, loop1, 0)

    macct = macc_ref[...]
    cnt_h = jnp.sum(macct, axis=0, keepdims=True)
    iota_bb0 = lax.broadcasted_iota(jnp.int32, (NB, NB), 0)
    iota_bb1 = lax.broadcasted_iota(jnp.int32, (NB, NB), 1)
    lt_bb = (iota_bb1 < iota_bb0).astype(jnp.float32)
    sh_col = jnp.sum(cnt_h * lt_bb, axis=1, keepdims=True)
    m_hi = jnp.floor(macct * (1.0 / 256.0))
    m_lo = macct - m_hi * 256.0
    m_hi_bf = m_hi.astype(jnp.bfloat16)
    m_lo_bf = m_lo.astype(jnp.bfloat16)

    gacc_ref[...] = jnp.zeros((NB, NB), jnp.float32)

    def loop2(q, carry):
        vrow = vrow_ref[pl.ds(q, 1), :]
        h_row = jnp.right_shift(vrow, 7)
        l_row = jnp.bitwise_and(vrow, 127)
        oht_bool = (h_row == iota_b_bP)
        oht_bf = oht_bool.astype(jnp.bfloat16)
        oht_f = oht_bool.astype(jnp.float32)
        a_row = jnp.sum(oht_f * sh_col, axis=0, keepdims=True)
        g = gacc_ref[...]
        g_hi = jnp.floor(g * (1.0 / 256.0))
        g_lo = g - g_hi * 256.0
        tblt = jnp.concatenate(
            [m_hi_bf, m_lo_bf,
             g_hi.astype(jnp.bfloat16), g_lo.astype(jnp.bfloat16)], axis=0)
        xyt = lax.dot_general(tblt, oht_bf, (((1,), (0,)), ((), ())),
                              preferred_element_type=jnp.float32)
        xt = xyt[0:NB, :] * 256.0 + xyt[NB:2 * NB, :]
        yt = xyt[2 * NB:3 * NB, :] * 256.0 + xyt[3 * NB:4 * NB, :]
        b_row = jnp.sum(jnp.where(iota_b_bP < l_row, xt, 0.0), axis=0,
                        keepdims=True)
        c1_row = jnp.sum(jnp.where(iota_b_bP == l_row, yt, 0.0), axis=0,
                         keepdims=True)
        rank = a_row + b_row + c1_row + c2_ref[pl.ds(q, 1), :]
        out_ref[pl.ds(q, 1), :] = rank.astype(jnp.int32)
        gacc_ref[...] += f2t_ref[pl.ds(q * NB, NB), :].astype(jnp.float32)
        return carry

    lax.fori_loop(0, Q, loop2, 0)


def _compute_rank(vrow):
    return pl.pallas_call(
        _rank_kernel,
        out_shape=jax.ShapeDtypeStruct((Q, P), jnp.int32),
        scratch_shapes=[
            pltpu.VMEM((Q * NB, NB), jnp.bfloat16),
            pltpu.VMEM((NB, NB), jnp.float32),
            pltpu.VMEM((NB, NB), jnp.float32),
            pltpu.VMEM((Q, P), jnp.float32),
        ],
    )(vrow)


NBUF = 3


def _sc_scatter(asc, cru, des, rank):
    mesh = plsc.VectorSubcoreMesh(core_axis_name="c", subcore_axis_name="s")

    @functools.partial(
        pl.kernel,
        out_type=jax.ShapeDtypeStruct((N, D), jnp.float32),
        mesh=mesh,
        scratch_types=(
            [pltpu.VMEM((NIT, CH), jnp.int32)]
            + [pltpu.VMEM((CH, D), jnp.float32)] * NBUF
            + [pltpu.SemaphoreType.DMA] * (2 * NBUF)
        ),
    )
    def scatter_kernel(asc_hbm, cru_hbm, des_hbm, rank_hbm, out_hbm,
                       idx_v, *bufs_sems):
        bufs = bufs_sems[:NBUF]
        lsems = bufs_sems[NBUF:2 * NBUF]
        ssems = bufs_sems[2 * NBUF:3 * NBUF]
        wid = lax.axis_index("s") * 2 + lax.axis_index("c")
        row0 = wid * RPW
        srcs = (asc_hbm, cru_hbm, des_hbm)
        idx_cps = []
        for s in range(NSRC):
            idx_cps.append(pltpu.async_copy(
                rank_hbm.at[pl.ds(s * (SRC_ROWS // CH) + wid * NCH, NCH)],
                idx_v.at[pl.ds(s * NCH, NCH)], lsems[0]))
        for cp in idx_cps:
            cp.wait()

        def start_load(i):
            s, k = divmod(i, NCH)
            return pltpu.async_copy(
                srcs[s].at[pl.ds(row0 + k * CH, CH)], bufs[i % NBUF],
                lsems[i % NBUF])

        loads = {i: start_load(i) for i in range(min(2, NIT))}
        stores = {}
        for i in range(NIT):
            loads[i].wait()
            stores[i] = pltpu.async_copy(
                bufs[i % NBUF], out_hbm.at[idx_v.at[i]], ssems[i % NBUF])
            m = i + 2
            if m < NIT:
                if m - NBUF >= 0:
                    stores[m - NBUF].wait()
                loads[m] = start_load(m)
        for i in range(NIT - NBUF, NIT):
            stores[i].wait()

    return scatter_kernel(asc, cru, des, rank)


def kernel(asc_dec, cru_dec, des_dec, concat_index):
    v = concat_index.astype(jnp.int32)
    vrow = v.reshape(Q, P)
    rank = _compute_rank(vrow)
    rank2d = rank.reshape(N // CH, CH)
    return _sc_scatter(asc_dec, cru_dec, des_dec, rank2d)


# speedup vs baseline: 1.7909x; 1.0255x over previous
import functools

import jax
import jax.numpy as jnp
from jax import lax
from jax.experimental import pallas as pl
from jax.experimental.pallas import tpu as pltpu
from jax.experimental.pallas import tpu_sc as plsc

N = 12288
NSRC = 3
SRC_ROWS = 4096
D = 1024
P = 512
Q = N // P
NB = 128

W = 32
RPW = SRC_ROWS // W
CH = 32
NCH = RPW // CH
NIT = NSRC * NCH


def _rank_kernel(vrow_ref, out_ref, f2t_ref, macc_ref, gacc_ref, c2_ref):
    iota_b_bP = lax.broadcasted_iota(jnp.int32, (NB, P), 0)
    iota_l_Pb = lax.broadcasted_iota(jnp.int32, (P, NB), 1)
    iota_p_PP = lax.broadcasted_iota(jnp.int32, (P, P), 0)
    iota_j_PP = lax.broadcasted_iota(jnp.int32, (P, P), 1)
    diag_PP = (iota_p_PP == iota_j_PP).astype(jnp.float32)

    macc_ref[...] = jnp.zeros((NB, NB), jnp.float32)

    def loop1(q, carry):
        vrow = vrow_ref[pl.ds(q * P, P)].reshape(1, P)
        vf = vrow.astype(jnp.float32)
        vcol = jnp.sum(vf * diag_PP, axis=1, keepdims=True).astype(jnp.int32)
        l_row = jnp.bitwise_and(vrow, 127)
        h_col = jnp.right_shift(vcol, 7)
        olt = (l_row == iota_b_bP).astype(jnp.bfloat16)
        oh = (h_col == iota_l_Pb).astype(jnp.bfloat16)
        f2t = lax.dot_general(olt, oh, (((1,), (0,)), ((), ())),
                              preferred_element_type=jnp.float32)
        macc_ref[...] += f2t
        f2t_ref[pl.ds(q * NB, NB), :] = f2t.astype(jnp.bfloat16)
        eq = (vcol == vrow) & (iota_j_PP < iota_p_PP)
        c2_col = jnp.sum(jnp.where(eq, 1.0, 0.0), axis=1, keepdims=True)
        c2_row = jnp.sum(c2_col * diag_PP, axis=0, keepdims=True)
        c2_ref[pl.ds(q, 1), :] = c2_row
        return carry

    lax.fori_loop(0, Q, loop1, 0)

    macct = macc_ref[...]
    cnt_h = jnp.sum(macct, axis=0, keepdims=True)
    iota_bb0 = lax.broadcasted_iota(jnp.int32, (NB, NB), 0)
    iota_bb1 = lax.broadcasted_iota(jnp.int32, (NB, NB), 1)
    lt_bb = (iota_bb1 < iota_bb0).astype(jnp.float32)
    sh_col = jnp.sum(cnt_h * lt_bb, axis=1, keepdims=True)
    m_hi = jnp.floor(macct * (1.0 / 256.0))
    m_lo = macct - m_hi * 256.0
    m_hi_bf = m_hi.astype(jnp.bfloat16)
    m_lo_bf = m_lo.astype(jnp.bfloat16)

    gacc_ref[...] = jnp.zeros((NB, NB), jnp.float32)

    def loop2(q, carry):
        vrow = vrow_ref[pl.ds(q * P, P)].reshape(1, P)
        h_row = jnp.right_shift(vrow, 7)
        l_row = jnp.bitwise_and(vrow, 127)
        oht_bool = (h_row == iota_b_bP)
        oht_bf = oht_bool.astype(jnp.bfloat16)
        oht_f = oht_bool.astype(jnp.float32)
        a_row = jnp.sum(oht_f * sh_col, axis=0, keepdims=True)
        g = gacc_ref[...]
        g_hi = jnp.floor(g * (1.0 / 256.0))
        g_lo = g - g_hi * 256.0
        tblt = jnp.concatenate(
            [m_hi_bf, m_lo_bf,
             g_hi.astype(jnp.bfloat16), g_lo.astype(jnp.bfloat16)], axis=0)
        xyt = lax.dot_general(tblt, oht_bf, (((1,), (0,)), ((), ())),
                              preferred_element_type=jnp.float32)
        xt = xyt[0:NB, :] * 256.0 + xyt[NB:2 * NB, :]
        yt = xyt[2 * NB:3 * NB, :] * 256.0 + xyt[3 * NB:4 * NB, :]
        b_row = jnp.sum(jnp.where(iota_b_bP < l_row, xt, 0.0), axis=0,
                        keepdims=True)
        c1_row = jnp.sum(jnp.where(iota_b_bP == l_row, yt, 0.0), axis=0,
                         keepdims=True)
        rank = a_row + b_row + c1_row + c2_ref[pl.ds(q, 1), :]
        out_ref[pl.ds(q, 1), :] = rank.astype(jnp.int32)
        gacc_ref[...] += f2t_ref[pl.ds(q * NB, NB), :].astype(jnp.float32)
        return carry

    lax.fori_loop(0, Q, loop2, 0)


def _compute_rank(vrow):
    return pl.pallas_call(
        _rank_kernel,
        out_shape=jax.ShapeDtypeStruct((Q, P), jnp.int32),
        scratch_shapes=[
            pltpu.VMEM((Q * NB, NB), jnp.bfloat16),
            pltpu.VMEM((NB, NB), jnp.float32),
            pltpu.VMEM((NB, NB), jnp.float32),
            pltpu.VMEM((Q, P), jnp.float32),
        ],
    )(vrow)


NBUF = 3


def _sc_scatter(asc, cru, des, rank):
    mesh = plsc.VectorSubcoreMesh(core_axis_name="c", subcore_axis_name="s")

    @functools.partial(
        pl.kernel,
        out_type=jax.ShapeDtypeStruct((N, D), jnp.float32),
        mesh=mesh,
        scratch_types=(
            [pltpu.VMEM((NIT, CH), jnp.int32),
             pltpu.VMEM((1, P), jnp.int32)]
            + [pltpu.VMEM((CH, D), jnp.float32)] * NBUF
            + [pltpu.SemaphoreType.DMA] * (2 * NBUF)
        ),
    )
    def scatter_kernel(asc_hbm, cru_hbm, des_hbm, rank_hbm, out_hbm,
                       idx_v, stage_v, *bufs_sems):
        bufs = bufs_sems[:NBUF]
        lsems = bufs_sems[NBUF:2 * NBUF]
        ssems = bufs_sems[2 * NBUF:3 * NBUF]
        wid = lax.axis_index("s") * 2 + lax.axis_index("c")
        row0 = wid * RPW
        srcs = (asc_hbm, cru_hbm, des_hbm)
        c0 = (wid % (P // RPW)) * RPW
        for s in range(NSRC):
            r = (s * SRC_ROWS + wid * RPW) // P
            pltpu.sync_copy(rank_hbm.at[pl.ds(r, 1)], stage_v)
            for k in range(NCH):
                for j in range(CH // 16):
                    idx_v[s * NCH + k, pl.ds(j * 16, 16)] = (
                        stage_v[0, pl.ds(c0 + k * CH + j * 16, 16)])

        def start_load(i):
            s, k = divmod(i, NCH)
            return pltpu.async_copy(
                srcs[s].at[pl.ds(row0 + k * CH, CH)], bufs[i % NBUF],
                lsems[i % NBUF])

        loads = {i: start_load(i) for i in range(min(2, NIT))}
        stores = {}
        for i in range(NIT):
            loads[i].wait()
            stores[i] = pltpu.async_copy(
                bufs[i % NBUF], out_hbm.at[idx_v.at[i]], ssems[i % NBUF])
            m = i + 2
            if m < NIT:
                if m - NBUF >= 0:
                    stores[m - NBUF].wait()
                loads[m] = start_load(m)
        for i in range(NIT - NBUF, NIT):
            stores[i].wait()

    return scatter_kernel(asc, cru, des, rank)


def kernel(asc_dec, cru_dec, des_dec, concat_index):
    v = concat_index.astype(jnp.int32)
    rank = _compute_rank(v)
    return _sc_scatter(asc_dec, cru_dec, des_dec, rank)


# speedup vs baseline: 1.9165x; 1.0702x over previous
import functools

import jax
import jax.numpy as jnp
from jax import lax
from jax.experimental import pallas as pl
from jax.experimental.pallas import tpu as pltpu
from jax.experimental.pallas import tpu_sc as plsc

N = 12288
NSRC = 3
SRC_ROWS = 4096
D = 1024
P = 512
Q = N // P
NB = 128

W = 32
RPW = SRC_ROWS // W
CH = 32
NCH = RPW // CH
NIT = NSRC * NCH


def _rank_kernel(vrow_ref, out_ref, f2t_ref, macc_ref, gacc_ref, c2_ref):
    iota_b_bP = lax.broadcasted_iota(jnp.int32, (NB, P), 0)
    iota_l_Pb = lax.broadcasted_iota(jnp.int32, (P, NB), 1)
    iota_p_PP = lax.broadcasted_iota(jnp.int32, (P, P), 0)
    iota_j_PP = lax.broadcasted_iota(jnp.int32, (P, P), 1)
    diag_PP = (iota_p_PP == iota_j_PP).astype(jnp.float32)

    macc_ref[...] = jnp.zeros((NB, NB), jnp.float32)

    def loop1(q, carry):
        vrow = vrow_ref[pl.ds(q * P, P)].reshape(1, P)
        vf = vrow.astype(jnp.float32)
        vcol = jnp.sum(vf * diag_PP, axis=1, keepdims=True).astype(jnp.int32)
        l_row = jnp.bitwise_and(vrow, 127)
        h_col = jnp.right_shift(vcol, 7)
        olt = (l_row == iota_b_bP).astype(jnp.bfloat16)
        oh = (h_col == iota_l_Pb).astype(jnp.bfloat16)
        f2t = lax.dot_general(olt, oh, (((1,), (0,)), ((), ())),
                              preferred_element_type=jnp.float32)
        macc_ref[...] += f2t
        f2t_ref[pl.ds(q * NB, NB), :] = f2t.astype(jnp.bfloat16)
        eq = (vcol == vrow) & (iota_j_PP < iota_p_PP)
        c2_col = jnp.sum(jnp.where(eq, 1.0, 0.0), axis=1, keepdims=True)
        c2_row = jnp.sum(c2_col * diag_PP, axis=0, keepdims=True)
        c2_ref[pl.ds(q, 1), :] = c2_row
        return carry

    lax.fori_loop(0, Q, loop1, 0, unroll=2)

    macct = macc_ref[...]
    cnt_h = jnp.sum(macct, axis=0, keepdims=True)
    iota_bb0 = lax.broadcasted_iota(jnp.int32, (NB, NB), 0)
    iota_bb1 = lax.broadcasted_iota(jnp.int32, (NB, NB), 1)
    lt_bb = (iota_bb1 < iota_bb0).astype(jnp.float32)
    sh_col = jnp.sum(cnt_h * lt_bb, axis=1, keepdims=True)
    m_hi = jnp.floor(macct * (1.0 / 256.0))
    m_lo = macct - m_hi * 256.0
    m_hi_bf = m_hi.astype(jnp.bfloat16)
    m_lo_bf = m_lo.astype(jnp.bfloat16)

    gacc_ref[...] = jnp.zeros((NB, NB), jnp.float32)

    def loop2(q, carry):
        vrow = vrow_ref[pl.ds(q * P, P)].reshape(1, P)
        h_row = jnp.right_shift(vrow, 7)
        l_row = jnp.bitwise_and(vrow, 127)
        oht_bool = (h_row == iota_b_bP)
        oht_bf = oht_bool.astype(jnp.bfloat16)
        oht_f = oht_bool.astype(jnp.float32)
        a_row = jnp.sum(oht_f * sh_col, axis=0, keepdims=True)
        g = gacc_ref[...]
        g_hi = jnp.floor(g * (1.0 / 256.0))
        g_lo = g - g_hi * 256.0
        tblt = jnp.concatenate(
            [m_hi_bf, m_lo_bf,
             g_hi.astype(jnp.bfloat16), g_lo.astype(jnp.bfloat16)], axis=0)
        xyt = lax.dot_general(tblt, oht_bf, (((1,), (0,)), ((), ())),
                              preferred_element_type=jnp.float32)
        xt = xyt[0:NB, :] * 256.0 + xyt[NB:2 * NB, :]
        yt = xyt[2 * NB:3 * NB, :] * 256.0 + xyt[3 * NB:4 * NB, :]
        b_row = jnp.sum(jnp.where(iota_b_bP < l_row, xt, 0.0), axis=0,
                        keepdims=True)
        c1_row = jnp.sum(jnp.where(iota_b_bP == l_row, yt, 0.0), axis=0,
                         keepdims=True)
        rank = a_row + b_row + c1_row + c2_ref[pl.ds(q, 1), :]
        out_ref[pl.ds(q, 1), :] = rank.astype(jnp.int32)
        gacc_ref[...] += f2t_ref[pl.ds(q * NB, NB), :].astype(jnp.float32)
        return carry

    lax.fori_loop(0, Q, loop2, 0, unroll=2)


def _compute_rank(vrow):
    return pl.pallas_call(
        _rank_kernel,
        out_shape=jax.ShapeDtypeStruct((Q, P), jnp.int32),
        scratch_shapes=[
            pltpu.VMEM((Q * NB, NB), jnp.bfloat16),
            pltpu.VMEM((NB, NB), jnp.float32),
            pltpu.VMEM((NB, NB), jnp.float32),
            pltpu.VMEM((Q, P), jnp.float32),
        ],
    )(vrow)


NBUF = 3


def _sc_scatter(asc, cru, des, rank):
    mesh = plsc.VectorSubcoreMesh(core_axis_name="c", subcore_axis_name="s")

    @functools.partial(
        pl.kernel,
        out_type=jax.ShapeDtypeStruct((N, D), jnp.float32),
        mesh=mesh,
        scratch_types=(
            [pltpu.VMEM((NIT, CH), jnp.int32),
             pltpu.VMEM((1, P), jnp.int32)]
            + [pltpu.VMEM((CH, D), jnp.float32)] * NBUF
            + [pltpu.SemaphoreType.DMA] * (2 * NBUF)
        ),
    )
    def scatter_kernel(asc_hbm, cru_hbm, des_hbm, rank_hbm, out_hbm,
                       idx_v, stage_v, *bufs_sems):
        bufs = bufs_sems[:NBUF]
        lsems = bufs_sems[NBUF:2 * NBUF]
        ssems = bufs_sems[2 * NBUF:3 * NBUF]
        wid = lax.axis_index("s") * 2 + lax.axis_index("c")
        row0 = wid * RPW
        srcs = (asc_hbm, cru_hbm, des_hbm)
        c0 = (wid % (P // RPW)) * RPW
        for s in range(NSRC):
            r = (s * SRC_ROWS + wid * RPW) // P
            pltpu.sync_copy(rank_hbm.at[pl.ds(r, 1)], stage_v)
            for k in range(NCH):
                for j in range(CH // 16):
                    idx_v[s * NCH + k, pl.ds(j * 16, 16)] = (
                        stage_v[0, pl.ds(c0 + k * CH + j * 16, 16)])

        def start_load(i):
            s, k = divmod(i, NCH)
            return pltpu.async_copy(
                srcs[s].at[pl.ds(row0 + k * CH, CH)], bufs[i % NBUF],
                lsems[i % NBUF])

        loads = {i: start_load(i) for i in range(min(2, NIT))}
        stores = {}
        for i in range(NIT):
            loads[i].wait()
            stores[i] = pltpu.async_copy(
                bufs[i % NBUF], out_hbm.at[idx_v.at[i]], ssems[i % NBUF])
            m = i + 2
            if m < NIT:
                if m - NBUF >= 0:
                    stores[m - NBUF].wait()
                loads[m] = start_load(m)
        for i in range(NIT - NBUF, NIT):
            stores[i].wait()

    return scatter_kernel(asc, cru, des, rank)


def kernel(asc_dec, cru_dec, des_dec, concat_index):
    v = concat_index.astype(jnp.int32)
    rank = _compute_rank(v)
    return _sc_scatter(asc_dec, cru_dec, des_dec, rank)


# speedup vs baseline: 1.9495x; 1.0172x over previous
import functools

import jax
import jax.numpy as jnp
from jax import lax
from jax.experimental import pallas as pl
from jax.experimental.pallas import tpu as pltpu
from jax.experimental.pallas import tpu_sc as plsc

N = 12288
NSRC = 3
SRC_ROWS = 4096
D = 1024
P = 512
Q = N // P
NB = 128

W = 32
RPW = SRC_ROWS // W
CH = 32
NCH = RPW // CH
NIT = NSRC * NCH


def _rank_kernel(vrow_ref, out_ref, f2t_ref, macc_ref, gacc_ref, c2_ref):
    iota_b_bP = lax.broadcasted_iota(jnp.int32, (NB, P), 0)
    iota_l_Pb = lax.broadcasted_iota(jnp.int32, (P, NB), 1)
    iota_p_PP = lax.broadcasted_iota(jnp.int32, (P, P), 0)
    iota_j_PP = lax.broadcasted_iota(jnp.int32, (P, P), 1)
    diag_PP = (iota_p_PP == iota_j_PP).astype(jnp.float32)

    macc_ref[...] = jnp.zeros((NB, NB), jnp.float32)

    def loop1(q, carry):
        vrow = vrow_ref[pl.ds(q * P, P)].reshape(1, P)
        vf = vrow.astype(jnp.float32)
        vcol = jnp.sum(vf * diag_PP, axis=1, keepdims=True).astype(jnp.int32)
        l_row = jnp.bitwise_and(vrow, 127)
        h_col = jnp.right_shift(vcol, 7)
        olt = (l_row == iota_b_bP).astype(jnp.bfloat16)
        oh = (h_col == iota_l_Pb).astype(jnp.bfloat16)
        f2t = lax.dot_general(olt, oh, (((1,), (0,)), ((), ())),
                              preferred_element_type=jnp.float32)
        macc_ref[...] += f2t
        f2t_ref[pl.ds(q * NB, NB), :] = f2t.astype(jnp.bfloat16)
        eq = (vcol == vrow) & (iota_j_PP < iota_p_PP)
        c2_col = jnp.sum(jnp.where(eq, 1.0, 0.0), axis=1, keepdims=True)
        c2_row = jnp.sum(c2_col * diag_PP, axis=0, keepdims=True)
        c2_ref[pl.ds(q, 1), :] = c2_row
        return carry

    lax.fori_loop(0, Q, loop1, 0, unroll=4)

    macct = macc_ref[...]
    cnt_h = jnp.sum(macct, axis=0, keepdims=True)
    iota_bb0 = lax.broadcasted_iota(jnp.int32, (NB, NB), 0)
    iota_bb1 = lax.broadcasted_iota(jnp.int32, (NB, NB), 1)
    lt_bb = (iota_bb1 < iota_bb0).astype(jnp.float32)
    sh_col = jnp.sum(cnt_h * lt_bb, axis=1, keepdims=True)
    m_hi = jnp.floor(macct * (1.0 / 256.0))
    m_lo = macct - m_hi * 256.0
    m_hi_bf = m_hi.astype(jnp.bfloat16)
    m_lo_bf = m_lo.astype(jnp.bfloat16)

    gacc_ref[...] = jnp.zeros((NB, NB), jnp.float32)

    def loop2(q, carry):
        vrow = vrow_ref[pl.ds(q * P, P)].reshape(1, P)
        h_row = jnp.right_shift(vrow, 7)
        l_row = jnp.bitwise_and(vrow, 127)
        oht_bool = (h_row == iota_b_bP)
        oht_bf = oht_bool.astype(jnp.bfloat16)
        oht_f = oht_bool.astype(jnp.float32)
        a_row = jnp.sum(oht_f * sh_col, axis=0, keepdims=True)
        g = gacc_ref[...]
        g_hi = jnp.floor(g * (1.0 / 256.0))
        g_lo = g - g_hi * 256.0
        tblt = jnp.concatenate(
            [m_hi_bf, m_lo_bf,
             g_hi.astype(jnp.bfloat16), g_lo.astype(jnp.bfloat16)], axis=0)
        xyt = lax.dot_general(tblt, oht_bf, (((1,), (0,)), ((), ())),
                              preferred_element_type=jnp.float32)
        xt = xyt[0:NB, :] * 256.0 + xyt[NB:2 * NB, :]
        yt = xyt[2 * NB:3 * NB, :] * 256.0 + xyt[3 * NB:4 * NB, :]
        b_row = jnp.sum(jnp.where(iota_b_bP < l_row, xt, 0.0), axis=0,
                        keepdims=True)
        c1_row = jnp.sum(jnp.where(iota_b_bP == l_row, yt, 0.0), axis=0,
                         keepdims=True)
        rank = a_row + b_row + c1_row + c2_ref[pl.ds(q, 1), :]
        out_ref[pl.ds(q, 1), :] = rank.astype(jnp.int32)
        gacc_ref[...] += f2t_ref[pl.ds(q * NB, NB), :].astype(jnp.float32)
        return carry

    lax.fori_loop(0, Q, loop2, 0, unroll=4)


def _compute_rank(vrow):
    return pl.pallas_call(
        _rank_kernel,
        out_shape=jax.ShapeDtypeStruct((Q, P), jnp.int32),
        scratch_shapes=[
            pltpu.VMEM((Q * NB, NB), jnp.bfloat16),
            pltpu.VMEM((NB, NB), jnp.float32),
            pltpu.VMEM((NB, NB), jnp.float32),
            pltpu.VMEM((Q, P), jnp.float32),
        ],
    )(vrow)


NBUF = 3


def _sc_scatter(asc, cru, des, rank):
    mesh = plsc.VectorSubcoreMesh(core_axis_name="c", subcore_axis_name="s")

    @functools.partial(
        pl.kernel,
        out_type=jax.ShapeDtypeStruct((N, D), jnp.float32),
        mesh=mesh,
        scratch_types=(
            [pltpu.VMEM((NIT, CH), jnp.int32),
             pltpu.VMEM((1, P), jnp.int32)]
            + [pltpu.VMEM((CH, D), jnp.float32)] * NBUF
            + [pltpu.SemaphoreType.DMA] * (2 * NBUF)
        ),
    )
    def scatter_kernel(asc_hbm, cru_hbm, des_hbm, rank_hbm, out_hbm,
                       idx_v, stage_v, *bufs_sems):
        bufs = bufs_sems[:NBUF]
        lsems = bufs_sems[NBUF:2 * NBUF]
        ssems = bufs_sems[2 * NBUF:3 * NBUF]
        wid = lax.axis_index("s") * 2 + lax.axis_index("c")
        row0 = wid * RPW
        srcs = (asc_hbm, cru_hbm, des_hbm)
        c0 = (wid % (P // RPW)) * RPW
        for s in range(NSRC):
            r = (s * SRC_ROWS + wid * RPW) // P
            pltpu.sync_copy(rank_hbm.at[pl.ds(r, 1)], stage_v)
            for k in range(NCH):
                for j in range(CH // 16):
                    idx_v[s * NCH + k, pl.ds(j * 16, 16)] = (
                        stage_v[0, pl.ds(c0 + k * CH + j * 16, 16)])

        def start_load(i):
            s, k = divmod(i, NCH)
            return pltpu.async_copy(
                srcs[s].at[pl.ds(row0 + k * CH, CH)], bufs[i % NBUF],
                lsems[i % NBUF])

        loads = {i: start_load(i) for i in range(min(2, NIT))}
        stores = {}
        for i in range(NIT):
            loads[i].wait()
            stores[i] = pltpu.async_copy(
                bufs[i % NBUF], out_hbm.at[idx_v.at[i]], ssems[i % NBUF])
            m = i + 2
            if m < NIT:
                if m - NBUF >= 0:
                    stores[m - NBUF].wait()
                loads[m] = start_load(m)
        for i in range(NIT - NBUF, NIT):
            stores[i].wait()

    return scatter_kernel(asc, cru, des, rank)


def kernel(asc_dec, cru_dec, des_dec, concat_index):
    v = concat_index.astype(jnp.int32)
    rank = _compute_rank(v)
    return _sc_scatter(asc_dec, cru_dec, des_dec, rank)


# speedup vs baseline: 1.9615x; 1.0062x over previous
import functools

import jax
import jax.numpy as jnp
from jax import lax
from jax.experimental import pallas as pl
from jax.experimental.pallas import tpu as pltpu
from jax.experimental.pallas import tpu_sc as plsc

N = 12288
NSRC = 3
SRC_ROWS = 4096
D = 1024
P = 512
Q = N // P
NB = 128

W = 32
RPW = SRC_ROWS // W
CH = 32
NCH = RPW // CH
NIT = NSRC * NCH


def _rank_kernel(vrow_ref, out_ref, f2t_ref, macc_ref, gacc_ref, c2_ref):
    iota_b_bP = lax.broadcasted_iota(jnp.int32, (NB, P), 0)
    iota_l_Pb = lax.broadcasted_iota(jnp.int32, (P, NB), 1)
    iota_p_PP = lax.broadcasted_iota(jnp.int32, (P, P), 0)
    iota_j_PP = lax.broadcasted_iota(jnp.int32, (P, P), 1)
    diag_PP = (iota_p_PP == iota_j_PP).astype(jnp.float32)

    macc_ref[...] = jnp.zeros((NB, NB), jnp.float32)

    def loop1(q, carry):
        vrow = vrow_ref[pl.ds(q * P, P)].reshape(1, P)
        vf = vrow.astype(jnp.float32)
        vcol = jnp.sum(vf * diag_PP, axis=1, keepdims=True).astype(jnp.int32)
        l_row = jnp.bitwise_and(vrow, 127)
        h_col = jnp.right_shift(vcol, 7)
        olt = (l_row == iota_b_bP).astype(jnp.bfloat16)
        oh = (h_col == iota_l_Pb).astype(jnp.bfloat16)
        f2t = lax.dot_general(olt, oh, (((1,), (0,)), ((), ())),
                              preferred_element_type=jnp.float32)
        macc_ref[...] += f2t
        f2t_ref[pl.ds(q * NB, NB), :] = f2t.astype(jnp.bfloat16)
        eq = (vcol == vrow) & (iota_j_PP < iota_p_PP)
        c2_col = jnp.sum(jnp.where(eq, 1.0, 0.0), axis=1, keepdims=True)
        c2_row = jnp.sum(c2_col * diag_PP, axis=0, keepdims=True)
        c2_ref[pl.ds(q, 1), :] = c2_row
        return carry

    lax.fori_loop(0, Q, loop1, 0, unroll=4)

    macct = macc_ref[...]
    cnt_h = jnp.sum(macct, axis=0, keepdims=True)
    iota_bb0 = lax.broadcasted_iota(jnp.int32, (NB, NB), 0)
    iota_bb1 = lax.broadcasted_iota(jnp.int32, (NB, NB), 1)
    lt_bb = (iota_bb0 < iota_bb1).astype(jnp.float32)
    diag_bb = (iota_bb0 == iota_bb1).astype(jnp.float32)
    cnt_col = jnp.sum(cnt_h * diag_bb, axis=1, keepdims=True)
    sh_row = jnp.sum(cnt_col * lt_bb, axis=0, keepdims=True)
    m_hi = jnp.floor(macct * (1.0 / 256.0)) * 256.0
    m_lo = macct - m_hi
    m_hi_bf = m_hi.astype(jnp.bfloat16)
    m_lo_bf = m_lo.astype(jnp.bfloat16)
    sh_hi = jnp.floor(sh_row * (1.0 / 256.0)) * 256.0
    sh_lo = sh_row - sh_hi
    sh_rows = jnp.concatenate(
        [sh_hi.astype(jnp.bfloat16), sh_lo.astype(jnp.bfloat16),
         jnp.zeros((6, NB), jnp.bfloat16)], axis=0)

    gacc_ref[...] = jnp.zeros((NB, NB), jnp.float32)

    def loop2(q, carry):
        vrow = vrow_ref[pl.ds(q * P, P)].reshape(1, P)
        h_row = jnp.right_shift(vrow, 7)
        l_row = jnp.bitwise_and(vrow, 127)
        oht_bf = (h_row == iota_b_bP).astype(jnp.bfloat16)
        g = gacc_ref[...]
        g_hi = jnp.floor(g * (1.0 / 256.0)) * 256.0
        g_lo = g - g_hi
        tblt = jnp.concatenate(
            [m_hi_bf, m_lo_bf,
             g_hi.astype(jnp.bfloat16), g_lo.astype(jnp.bfloat16),
             sh_rows], axis=0)
        xyt = lax.dot_general(tblt, oht_bf, (((1,), (0,)), ((), ())),
                              preferred_element_type=jnp.float32)
        xt = xyt[0:NB, :] + xyt[NB:2 * NB, :]
        yt = xyt[2 * NB:3 * NB, :] + xyt[3 * NB:4 * NB, :]
        a_row = xyt[4 * NB:4 * NB + 1, :] + xyt[4 * NB + 1:4 * NB + 2, :]
        b_row = jnp.sum(jnp.where(iota_b_bP < l_row, xt, 0.0), axis=0,
                        keepdims=True)
        c1_row = jnp.sum(jnp.where(iota_b_bP == l_row, yt, 0.0), axis=0,
                         keepdims=True)
        rank = a_row + b_row + c1_row + c2_ref[pl.ds(q, 1), :]
        out_ref[pl.ds(q, 1), :] = rank.astype(jnp.int32)
        gacc_ref[...] += f2t_ref[pl.ds(q * NB, NB), :].astype(jnp.float32)
        return carry

    lax.fori_loop(0, Q, loop2, 0, unroll=4)


def _compute_rank(vrow):
    return pl.pallas_call(
        _rank_kernel,
        out_shape=jax.ShapeDtypeStruct((Q, P), jnp.int32),
        scratch_shapes=[
            pltpu.VMEM((Q * NB, NB), jnp.bfloat16),
            pltpu.VMEM((NB, NB), jnp.float32),
            pltpu.VMEM((NB, NB), jnp.float32),
            pltpu.VMEM((Q, P), jnp.float32),
        ],
    )(vrow)


NBUF = 3


def _sc_scatter(asc, cru, des, rank):
    mesh = plsc.VectorSubcoreMesh(core_axis_name="c", subcore_axis_name="s")

    @functools.partial(
        pl.kernel,
        out_type=jax.ShapeDtypeStruct((N, D), jnp.float32),
        mesh=mesh,
        scratch_types=(
            [pltpu.VMEM((NIT, CH), jnp.int32),
             pltpu.VMEM((1, P), jnp.int32)]
            + [pltpu.VMEM((CH, D), jnp.float32)] * NBUF
            + [pltpu.SemaphoreType.DMA] * (2 * NBUF)
        ),
    )
    def scatter_kernel(asc_hbm, cru_hbm, des_hbm, rank_hbm, out_hbm,
                       idx_v, stage_v, *bufs_sems):
        bufs = bufs_sems[:NBUF]
        lsems = bufs_sems[NBUF:2 * NBUF]
        ssems = bufs_sems[2 * NBUF:3 * NBUF]
        wid = lax.axis_index("s") * 2 + lax.axis_index("c")
        row0 = wid * RPW
        srcs = (asc_hbm, cru_hbm, des_hbm)
        c0 = (wid % (P // RPW)) * RPW
        for s in range(NSRC):
            r = (s * SRC_ROWS + wid * RPW) // P
            pltpu.sync_copy(rank_hbm.at[pl.ds(r, 1)], stage_v)
            for k in range(NCH):
                for j in range(CH // 16):
                    idx_v[s * NCH + k, pl.ds(j * 16, 16)] = (
                        stage_v[0, pl.ds(c0 + k * CH + j * 16, 16)])

        def start_load(i):
            s, k = divmod(i, NCH)
            return pltpu.async_copy(
                srcs[s].at[pl.ds(row0 + k * CH, CH)], bufs[i % NBUF],
                lsems[i % NBUF])

        loads = {i: start_load(i) for i in range(min(2, NIT))}
        stores = {}
        for i in range(NIT):
            loads[i].wait()
            stores[i] = pltpu.async_copy(
                bufs[i % NBUF], out_hbm.at[idx_v.at[i]], ssems[i % NBUF])
            m = i + 2
            if m < NIT:
                if m - NBUF >= 0:
                    stores[m - NBUF].wait()
                loads[m] = start_load(m)
        for i in range(NIT - NBUF, NIT):
            stores[i].wait()

    return scatter_kernel(asc, cru, des, rank)


def kernel(asc_dec, cru_dec, des_dec, concat_index):
    v = concat_index.astype(jnp.int32)
    rank = _compute_rank(v)
    return _sc_scatter(asc_dec, cru_dec, des_dec, rank)


# speedup vs baseline: 1.9777x; 1.0083x over previous
import functools

import jax
import jax.numpy as jnp
from jax import lax
from jax.experimental import pallas as pl
from jax.experimental.pallas import tpu as pltpu
from jax.experimental.pallas import tpu_sc as plsc

N = 12288
NSRC = 3
SRC_ROWS = 4096
D = 1024
P = 512
Q = N // P
NB = 128

W = 32
RPW = SRC_ROWS // W
CH = 32
NCH = RPW // CH
NIT = NSRC * NCH


def _rank_kernel(vrow_ref, out_ref, f2t_ref, macc_ref, gacc_ref, c2_ref):
    iota_b_bP = lax.broadcasted_iota(jnp.int32, (NB, P), 0)
    iota_l_Pb = lax.broadcasted_iota(jnp.int32, (P, NB), 1)
    iota_p_PP = lax.broadcasted_iota(jnp.int32, (P, P), 0)
    iota_j_PP = lax.broadcasted_iota(jnp.int32, (P, P), 1)
    diag_PP = (iota_p_PP == iota_j_PP).astype(jnp.float32)

    macc_ref[...] = jnp.zeros((NB, NB), jnp.float32)

    def loop1(q, carry):
        vrow = vrow_ref[pl.ds(q * P, P)].reshape(1, P)
        vf = vrow.astype(jnp.float32)
        vcol = jnp.sum(vf * diag_PP, axis=1, keepdims=True).astype(jnp.int32)
        l_row = jnp.bitwise_and(vrow, 127)
        h_col = jnp.right_shift(vcol, 7)
        olt = (l_row == iota_b_bP).astype(jnp.bfloat16)
        oh = (h_col == iota_l_Pb).astype(jnp.bfloat16)
        f2t = lax.dot_general(olt, oh, (((1,), (0,)), ((), ())),
                              preferred_element_type=jnp.float32)
        macc_ref[...] += f2t
        f2t_ref[pl.ds(q * NB, NB), :] = f2t.astype(jnp.bfloat16)
        eq = (vcol == vrow) & (iota_j_PP < iota_p_PP)
        c2_col = jnp.sum(jnp.where(eq, 1.0, 0.0), axis=1, keepdims=True)
        c2_row = jnp.sum(c2_col * diag_PP, axis=0, keepdims=True)
        c2_ref[pl.ds(q, 1), :] = c2_row
        return carry

    lax.fori_loop(0, Q, loop1, 0, unroll=8)

    macct = macc_ref[...]
    cnt_h = jnp.sum(macct, axis=0, keepdims=True)
    iota_bb0 = lax.broadcasted_iota(jnp.int32, (NB, NB), 0)
    iota_bb1 = lax.broadcasted_iota(jnp.int32, (NB, NB), 1)
    lt_bb = (iota_bb0 < iota_bb1).astype(jnp.float32)
    diag_bb = (iota_bb0 == iota_bb1).astype(jnp.float32)
    cnt_col = jnp.sum(cnt_h * diag_bb, axis=1, keepdims=True)
    sh_row = jnp.sum(cnt_col * lt_bb, axis=0, keepdims=True)
    m_hi = jnp.floor(macct * (1.0 / 256.0)) * 256.0
    m_lo = macct - m_hi
    m_hi_bf = m_hi.astype(jnp.bfloat16)
    m_lo_bf = m_lo.astype(jnp.bfloat16)
    sh_hi = jnp.floor(sh_row * (1.0 / 256.0)) * 256.0
    sh_lo = sh_row - sh_hi
    sh_rows = jnp.concatenate(
        [sh_hi.astype(jnp.bfloat16), sh_lo.astype(jnp.bfloat16),
         jnp.zeros((6, NB), jnp.bfloat16)], axis=0)

    gacc_ref[...] = jnp.zeros((NB, NB), jnp.float32)

    def loop2(q, carry):
        vrow = vrow_ref[pl.ds(q * P, P)].reshape(1, P)
        h_row = jnp.right_shift(vrow, 7)
        l_row = jnp.bitwise_and(vrow, 127)
        oht_bf = (h_row == iota_b_bP).astype(jnp.bfloat16)
        g = gacc_ref[...]
        g_hi = jnp.floor(g * (1.0 / 256.0)) * 256.0
        g_lo = g - g_hi
        tblt = jnp.concatenate(
            [m_hi_bf, m_lo_bf,
             g_hi.astype(jnp.bfloat16), g_lo.astype(jnp.bfloat16),
             sh_rows], axis=0)
        xyt = lax.dot_general(tblt, oht_bf, (((1,), (0,)), ((), ())),
                              preferred_element_type=jnp.float32)
        xt = xyt[0:NB, :] + xyt[NB:2 * NB, :]
        yt = xyt[2 * NB:3 * NB, :] + xyt[3 * NB:4 * NB, :]
        a_row = xyt[4 * NB:4 * NB + 1, :] + xyt[4 * NB + 1:4 * NB + 2, :]
        b_row = jnp.sum(jnp.where(iota_b_bP < l_row, xt, 0.0), axis=0,
                        keepdims=True)
        c1_row = jnp.sum(jnp.where(iota_b_bP == l_row, yt, 0.0), axis=0,
                         keepdims=True)
        rank = a_row + b_row + c1_row + c2_ref[pl.ds(q, 1), :]
        out_ref[pl.ds(q, 1), :] = rank.astype(jnp.int32)
        gacc_ref[...] += f2t_ref[pl.ds(q * NB, NB), :].astype(jnp.float32)
        return carry

    lax.fori_loop(0, Q, loop2, 0, unroll=8)


def _compute_rank(vrow):
    return pl.pallas_call(
        _rank_kernel,
        out_shape=jax.ShapeDtypeStruct((Q, P), jnp.int32),
        scratch_shapes=[
            pltpu.VMEM((Q * NB, NB), jnp.bfloat16),
            pltpu.VMEM((NB, NB), jnp.float32),
            pltpu.VMEM((NB, NB), jnp.float32),
            pltpu.VMEM((Q, P), jnp.float32),
        ],
    )(vrow)


NBUF = 3


def _sc_scatter(asc, cru, des, rank):
    mesh = plsc.VectorSubcoreMesh(core_axis_name="c", subcore_axis_name="s")

    @functools.partial(
        pl.kernel,
        out_type=jax.ShapeDtypeStruct((N, D), jnp.float32),
        mesh=mesh,
        scratch_types=(
            [pltpu.VMEM((NIT, CH), jnp.int32),
             pltpu.VMEM((1, P), jnp.int32)]
            + [pltpu.VMEM((CH, D), jnp.float32)] * NBUF
            + [pltpu.SemaphoreType.DMA] * (2 * NBUF)
        ),
    )
    def scatter_kernel(asc_hbm, cru_hbm, des_hbm, rank_hbm, out_hbm,
                       idx_v, stage_v, *bufs_sems):
        bufs = bufs_sems[:NBUF]
        lsems = bufs_sems[NBUF:2 * NBUF]
        ssems = bufs_sems[2 * NBUF:3 * NBUF]
        wid = lax.axis_index("s") * 2 + lax.axis_index("c")
        row0 = wid * RPW
        srcs = (asc_hbm, cru_hbm, des_hbm)
        c0 = (wid % (P // RPW)) * RPW
        for s in range(NSRC):
            r = (s * SRC_ROWS + wid * RPW) // P
            pltpu.sync_copy(rank_hbm.at[pl.ds(r, 1)], stage_v)
            for k in range(NCH):
                for j in range(CH // 16):
                    idx_v[s * NCH + k, pl.ds(j * 16, 16)] = (
                        stage_v[0, pl.ds(c0 + k * CH + j * 16, 16)])

        def start_load(i):
            s, k = divmod(i, NCH)
            return pltpu.async_copy(
                srcs[s].at[pl.ds(row0 + k * CH, CH)], bufs[i % NBUF],
                lsems[i % NBUF])

        loads = {i: start_load(i) for i in range(min(2, NIT))}
        stores = {}
        for i in range(NIT):
            loads[i].wait()
            stores[i] = pltpu.async_copy(
                bufs[i % NBUF], out_hbm.at[idx_v.at[i]], ssems[i % NBUF])
            m = i + 2
            if m < NIT:
                if m - NBUF >= 0:
                    stores[m - NBUF].wait()
                loads[m] = start_load(m)
        for i in range(NIT - NBUF, NIT):
            stores[i].wait()

    return scatter_kernel(asc, cru, des, rank)


def kernel(asc_dec, cru_dec, des_dec, concat_index):
    v = concat_index.astype(jnp.int32)
    rank = _compute_rank(v)
    return _sc_scatter(asc_dec, cru_dec, des_dec, rank)


# speedup vs baseline: 1.9891x; 1.0057x over previous
import functools

import jax
import jax.numpy as jnp
from jax import lax
from jax.experimental import pallas as pl
from jax.experimental.pallas import tpu as pltpu
from jax.experimental.pallas import tpu_sc as plsc

N = 12288
NSRC = 3
SRC_ROWS = 4096
D = 1024
P = 512
Q = N // P
NB = 128

W = 32
RPW = SRC_ROWS // W
CH = 32
NCH = RPW // CH
NIT = NSRC * NCH


def _rank_kernel(vrow_ref, out_ref, f2t_ref, macc_ref, gacc_ref, c2_ref):
    iota_b_bP = lax.broadcasted_iota(jnp.int32, (NB, P), 0)
    iota_l_Pb = lax.broadcasted_iota(jnp.int32, (P, NB), 1)
    iota_p_PP = lax.broadcasted_iota(jnp.int32, (P, P), 0)
    iota_j_PP = lax.broadcasted_iota(jnp.int32, (P, P), 1)
    diag_PP = (iota_p_PP == iota_j_PP).astype(jnp.float32)

    macc_ref[...] = jnp.zeros((NB, NB), jnp.float32)

    def loop1(q, carry):
        vrow = vrow_ref[pl.ds(q * P, P)].reshape(1, P)
        vf = vrow.astype(jnp.float32)
        vcol = jnp.sum(vf * diag_PP, axis=1, keepdims=True).astype(jnp.int32)
        l_row = jnp.bitwise_and(vrow, 127)
        h_col = jnp.right_shift(vcol, 7)
        olt = (l_row == iota_b_bP).astype(jnp.bfloat16)
        oh = (h_col == iota_l_Pb).astype(jnp.bfloat16)
        f2t = lax.dot_general(olt, oh, (((1,), (0,)), ((), ())),
                              preferred_element_type=jnp.float32)
        macc_ref[...] += f2t
        f2t_ref[pl.ds(q * NB, NB), :] = f2t.astype(jnp.bfloat16)
        eq = (vcol == vrow) & (iota_j_PP < iota_p_PP)
        c2_col = jnp.sum(jnp.where(eq, 1.0, 0.0), axis=1, keepdims=True)
        c2_row = jnp.sum(c2_col * diag_PP, axis=0, keepdims=True)
        c2_ref[pl.ds(q, 1), :] = c2_row
        return carry

    lax.fori_loop(0, Q, loop1, 0, unroll=12)

    macct = macc_ref[...]
    cnt_h = jnp.sum(macct, axis=0, keepdims=True)
    iota_bb0 = lax.broadcasted_iota(jnp.int32, (NB, NB), 0)
    iota_bb1 = lax.broadcasted_iota(jnp.int32, (NB, NB), 1)
    lt_bb = (iota_bb0 < iota_bb1).astype(jnp.float32)
    diag_bb = (iota_bb0 == iota_bb1).astype(jnp.float32)
    cnt_col = jnp.sum(cnt_h * diag_bb, axis=1, keepdims=True)
    sh_row = jnp.sum(cnt_col * lt_bb, axis=0, keepdims=True)
    m_hi = jnp.floor(macct * (1.0 / 256.0)) * 256.0
    m_lo = macct - m_hi
    m_hi_bf = m_hi.astype(jnp.bfloat16)
    m_lo_bf = m_lo.astype(jnp.bfloat16)
    sh_hi = jnp.floor(sh_row * (1.0 / 256.0)) * 256.0
    sh_lo = sh_row - sh_hi
    sh_rows = jnp.concatenate(
        [sh_hi.astype(jnp.bfloat16), sh_lo.astype(jnp.bfloat16),
         jnp.zeros((6, NB), jnp.bfloat16)], axis=0)

    gacc_ref[...] = jnp.zeros((NB, NB), jnp.float32)

    def loop2(q, carry):
        vrow = vrow_ref[pl.ds(q * P, P)].reshape(1, P)
        h_row = jnp.right_shift(vrow, 7)
        l_row = jnp.bitwise_and(vrow, 127)
        oht_bf = (h_row == iota_b_bP).astype(jnp.bfloat16)
        g = gacc_ref[...]
        g_hi = jnp.floor(g * (1.0 / 256.0)) * 256.0
        g_lo = g - g_hi
        tblt = jnp.concatenate(
            [m_hi_bf, m_lo_bf,
             g_hi.astype(jnp.bfloat16), g_lo.astype(jnp.bfloat16),
             sh_rows], axis=0)
        xyt = lax.dot_general(tblt, oht_bf, (((1,), (0,)), ((), ())),
                              preferred_element_type=jnp.float32)
        xt = xyt[0:NB, :] + xyt[NB:2 * NB, :]
        yt = xyt[2 * NB:3 * NB, :] + xyt[3 * NB:4 * NB, :]
        a_row = xyt[4 * NB:4 * NB + 1, :] + xyt[4 * NB + 1:4 * NB + 2, :]
        b_row = jnp.sum(jnp.where(iota_b_bP < l_row, xt, 0.0), axis=0,
                        keepdims=True)
        c1_row = jnp.sum(jnp.where(iota_b_bP == l_row, yt, 0.0), axis=0,
                         keepdims=True)
        rank = a_row + b_row + c1_row + c2_ref[pl.ds(q, 1), :]
        out_ref[pl.ds(q, 1), :] = rank.astype(jnp.int32)
        gacc_ref[...] += f2t_ref[pl.ds(q * NB, NB), :].astype(jnp.float32)
        return carry

    lax.fori_loop(0, Q, loop2, 0, unroll=12)


def _compute_rank(vrow):
    return pl.pallas_call(
        _rank_kernel,
        out_shape=jax.ShapeDtypeStruct((Q, P), jnp.int32),
        scratch_shapes=[
            pltpu.VMEM((Q * NB, NB), jnp.bfloat16),
            pltpu.VMEM((NB, NB), jnp.float32),
            pltpu.VMEM((NB, NB), jnp.float32),
            pltpu.VMEM((Q, P), jnp.float32),
        ],
    )(vrow)


NBUF = 3


def _sc_scatter(asc, cru, des, rank):
    mesh = plsc.VectorSubcoreMesh(core_axis_name="c", subcore_axis_name="s")

    @functools.partial(
        pl.kernel,
        out_type=jax.ShapeDtypeStruct((N, D), jnp.float32),
        mesh=mesh,
        scratch_types=(
            [pltpu.VMEM((NIT, CH), jnp.int32),
             pltpu.VMEM((1, P), jnp.int32)]
            + [pltpu.VMEM((CH, D), jnp.float32)] * NBUF
            + [pltpu.SemaphoreType.DMA] * (2 * NBUF)
        ),
    )
    def scatter_kernel(asc_hbm, cru_hbm, des_hbm, rank_hbm, out_hbm,
                       idx_v, stage_v, *bufs_sems):
        bufs = bufs_sems[:NBUF]
        lsems = bufs_sems[NBUF:2 * NBUF]
        ssems = bufs_sems[2 * NBUF:3 * NBUF]
        wid = lax.axis_index("s") * 2 + lax.axis_index("c")
        row0 = wid * RPW
        srcs = (asc_hbm, cru_hbm, des_hbm)
        c0 = (wid % (P // RPW)) * RPW
        for s in range(NSRC):
            r = (s * SRC_ROWS + wid * RPW) // P
            pltpu.sync_copy(rank_hbm.at[pl.ds(r, 1)], stage_v)
            for k in range(NCH):
                for j in range(CH // 16):
                    idx_v[s * NCH + k, pl.ds(j * 16, 16)] = (
                        stage_v[0, pl.ds(c0 + k * CH + j * 16, 16)])

        def start_load(i):
            s, k = divmod(i, NCH)
            return pltpu.async_copy(
                srcs[s].at[pl.ds(row0 + k * CH, CH)], bufs[i % NBUF],
                lsems[i % NBUF])

        loads = {i: start_load(i) for i in range(min(2, NIT))}
        stores = {}
        for i in range(NIT):
            loads[i].wait()
            stores[i] = pltpu.async_copy(
                bufs[i % NBUF], out_hbm.at[idx_v.at[i]], ssems[i % NBUF])
            m = i + 2
            if m < NIT:
                if m - NBUF >= 0:
                    stores[m - NBUF].wait()
                loads[m] = start_load(m)
        for i in range(NIT - NBUF, NIT):
            stores[i].wait()

    return scatter_kernel(asc, cru, des, rank)


def kernel(asc_dec, cru_dec, des_dec, concat_index):
    v = concat_index.astype(jnp.int32)
    rank = _compute_rank(v)
    return _sc_scatter(asc_dec, cru_dec, des_dec, rank)


# speedup vs baseline: 2.0033x; 1.0072x over previous
import functools

import jax
import jax.numpy as jnp
from jax import lax
from jax.experimental import pallas as pl
from jax.experimental.pallas import tpu as pltpu
from jax.experimental.pallas import tpu_sc as plsc

N = 12288
NSRC = 3
SRC_ROWS = 4096
D = 1024
P = 512
Q = N // P
NB = 128

W = 32
RPW = SRC_ROWS // W
CH = 32
NCH = RPW // CH
NIT = NSRC * NCH


def _rank_kernel(vrow_ref, out_ref, f2t_ref, macc_ref, gacc_ref, c2_ref):
    iota_b_bP = lax.broadcasted_iota(jnp.int32, (NB, P), 0)
    iota_l_Pb = lax.broadcasted_iota(jnp.int32, (P, NB), 1)
    iota_p_PP = lax.broadcasted_iota(jnp.int32, (P, P), 0)
    iota_j_PP = lax.broadcasted_iota(jnp.int32, (P, P), 1)
    diag_PP = (iota_p_PP == iota_j_PP).astype(jnp.float32)

    macc_ref[...] = jnp.zeros((NB, NB), jnp.float32)

    def loop1(q, carry):
        vrow = vrow_ref[pl.ds(q * P, P)].reshape(1, P)
        vf = vrow.astype(jnp.float32)
        vcol = jnp.sum(vf * diag_PP, axis=1, keepdims=True).astype(jnp.int32)
        l_row = jnp.bitwise_and(vrow, 127)
        h_col = jnp.right_shift(vcol, 7)
        olt = (l_row == iota_b_bP).astype(jnp.bfloat16)
        oh = (h_col == iota_l_Pb).astype(jnp.bfloat16)
        f2t = lax.dot_general(olt, oh, (((1,), (0,)), ((), ())),
                              preferred_element_type=jnp.float32)
        macc_ref[...] += f2t
        f2t_ref[pl.ds(q * NB, NB), :] = f2t.astype(jnp.bfloat16)
        eq = (vcol == vrow) & (iota_j_PP < iota_p_PP)
        c2_col = jnp.sum(jnp.where(eq, 1.0, 0.0), axis=1, keepdims=True)
        c2_row = jnp.sum(c2_col * diag_PP, axis=0, keepdims=True)
        c2_ref[pl.ds(q, 1), :] = c2_row
        return carry

    lax.fori_loop(0, Q, loop1, 0, unroll=24)

    macct = macc_ref[...]
    cnt_h = jnp.sum(macct, axis=0, keepdims=True)
    iota_bb0 = lax.broadcasted_iota(jnp.int32, (NB, NB), 0)
    iota_bb1 = lax.broadcasted_iota(jnp.int32, (NB, NB), 1)
    lt_bb = (iota_bb0 < iota_bb1).astype(jnp.float32)
    diag_bb = (iota_bb0 == iota_bb1).astype(jnp.float32)
    cnt_col = jnp.sum(cnt_h * diag_bb, axis=1, keepdims=True)
    sh_row = jnp.sum(cnt_col * lt_bb, axis=0, keepdims=True)
    m_hi = jnp.floor(macct * (1.0 / 256.0)) * 256.0
    m_lo = macct - m_hi
    m_hi_bf = m_hi.astype(jnp.bfloat16)
    m_lo_bf = m_lo.astype(jnp.bfloat16)
    sh_hi = jnp.floor(sh_row * (1.0 / 256.0)) * 256.0
    sh_lo = sh_row - sh_hi
    sh_rows = jnp.concatenate(
        [sh_hi.astype(jnp.bfloat16), sh_lo.astype(jnp.bfloat16),
         jnp.zeros((6, NB), jnp.bfloat16)], axis=0)

    gacc_ref[...] = jnp.zeros((NB, NB), jnp.float32)

    def loop2(q, carry):
        vrow = vrow_ref[pl.ds(q * P, P)].reshape(1, P)
        h_row = jnp.right_shift(vrow, 7)
        l_row = jnp.bitwise_and(vrow, 127)
        oht_bf = (h_row == iota_b_bP).astype(jnp.bfloat16)
        g = gacc_ref[...]
        g_hi = jnp.floor(g * (1.0 / 256.0)) * 256.0
        g_lo = g - g_hi
        tblt = jnp.concatenate(
            [m_hi_bf, m_lo_bf,
             g_hi.astype(jnp.bfloat16), g_lo.astype(jnp.bfloat16),
             sh_rows], axis=0)
        xyt = lax.dot_general(tblt, oht_bf, (((1,), (0,)), ((), ())),
                              preferred_element_type=jnp.float32)
        xt = xyt[0:NB, :] + xyt[NB:2 * NB, :]
        yt = xyt[2 * NB:3 * NB, :] + xyt[3 * NB:4 * NB, :]
        a_row = xyt[4 * NB:4 * NB + 1, :] + xyt[4 * NB + 1:4 * NB + 2, :]
        b_row = jnp.sum(jnp.where(iota_b_bP < l_row, xt, 0.0), axis=0,
                        keepdims=True)
        c1_row = jnp.sum(jnp.where(iota_b_bP == l_row, yt, 0.0), axis=0,
                         keepdims=True)
        rank = a_row + b_row + c1_row + c2_ref[pl.ds(q, 1), :]
        out_ref[pl.ds(q, 1), :] = rank.astype(jnp.int32)
        gacc_ref[...] += f2t_ref[pl.ds(q * NB, NB), :].astype(jnp.float32)
        return carry

    lax.fori_loop(0, Q, loop2, 0, unroll=24)


def _compute_rank(vrow):
    return pl.pallas_call(
        _rank_kernel,
        out_shape=jax.ShapeDtypeStruct((Q, P), jnp.int32),
        scratch_shapes=[
            pltpu.VMEM((Q * NB, NB), jnp.bfloat16),
            pltpu.VMEM((NB, NB), jnp.float32),
            pltpu.VMEM((NB, NB), jnp.float32),
            pltpu.VMEM((Q, P), jnp.float32),
        ],
    )(vrow)


NBUF = 3


def _sc_scatter(asc, cru, des, rank):
    mesh = plsc.VectorSubcoreMesh(core_axis_name="c", subcore_axis_name="s")

    @functools.partial(
        pl.kernel,
        out_type=jax.ShapeDtypeStruct((N, D), jnp.float32),
        mesh=mesh,
        scratch_types=(
            [pltpu.VMEM((NIT, CH), jnp.int32),
             pltpu.VMEM((1, P), jnp.int32)]
            + [pltpu.VMEM((CH, D), jnp.float32)] * NBUF
            + [pltpu.SemaphoreType.DMA] * (2 * NBUF)
        ),
    )
    def scatter_kernel(asc_hbm, cru_hbm, des_hbm, rank_hbm, out_hbm,
                       idx_v, stage_v, *bufs_sems):
        bufs = bufs_sems[:NBUF]
        lsems = bufs_sems[NBUF:2 * NBUF]
        ssems = bufs_sems[2 * NBUF:3 * NBUF]
        wid = lax.axis_index("s") * 2 + lax.axis_index("c")
        row0 = wid * RPW
        srcs = (asc_hbm, cru_hbm, des_hbm)
        c0 = (wid % (P // RPW)) * RPW
        for s in range(NSRC):
            r = (s * SRC_ROWS + wid * RPW) // P
            pltpu.sync_copy(rank_hbm.at[pl.ds(r, 1)], stage_v)
            for k in range(NCH):
                for j in range(CH // 16):
                    idx_v[s * NCH + k, pl.ds(j * 16, 16)] = (
                        stage_v[0, pl.ds(c0 + k * CH + j * 16, 16)])

        def start_load(i):
            s, k = divmod(i, NCH)
            return pltpu.async_copy(
                srcs[s].at[pl.ds(row0 + k * CH, CH)], bufs[i % NBUF],
                lsems[i % NBUF])

        loads = {i: start_load(i) for i in range(min(2, NIT))}
        stores = {}
        for i in range(NIT):
            loads[i].wait()
            stores[i] = pltpu.async_copy(
                bufs[i % NBUF], out_hbm.at[idx_v.at[i]], ssems[i % NBUF])
            m = i + 2
            if m < NIT:
                if m - NBUF >= 0:
                    stores[m - NBUF].wait()
                loads[m] = start_load(m)
        for i in range(NIT - NBUF, NIT):
            stores[i].wait()

    return scatter_kernel(asc, cru, des, rank)


def kernel(asc_dec, cru_dec, des_dec, concat_index):
    v = concat_index.astype(jnp.int32)
    rank = _compute_rank(v)
    return _sc_scatter(asc_dec, cru_dec, des_dec, rank)
